# Initial kernel scaffold; baseline (speedup 1.0000x reference)
import jax
import jax.numpy as jnp
from jax.experimental import pallas as pl


def kernel(features, edge_index, edge_values, index):
    raise NotImplementedError("write your pallas kernel here")


# speedup vs baseline: 3.1541x; 3.1541x over previous
import functools

import jax
import jax.numpy as jnp
from jax import lax
from jax.experimental import pallas as pl
from jax.experimental.pallas import tpu as pltpu
from jax.experimental.pallas import tpu_sc as plsc

N = 10000
E = 160000
D = 256

NC = 2
NS = 16
L = 16
HALF = N // NC
NBLK8 = HALF // 8
ACC_ROWS = 8 * ((NBLK8 + NS - 1) // NS)
SB = 2000
EPT = E // NS
FB = 80
CAP = FB + L


def _nr_rsqrt(x):
    i = lax.bitcast_convert_type(x, jnp.int32)
    i = jnp.int32(0x5F3759DF) - jnp.right_shift(i, 1)
    y = lax.bitcast_convert_type(i, jnp.float32)
    xh = x * jnp.float32(0.5)
    for _ in range(3):
        y = y * (jnp.float32(1.5) - xh * y * y)
    return y


def _body(feat_hbm, row_hbm, col_hbm, ev_hbm, out_hbm,
          hist_sh, rs_sh, hist_v, acc, gbuf,
          rbuf, cbuf, ebuf,
          colb, tgtb, cofb, tmp, tmpv, sem):
    c = lax.axis_index("c")
    s = lax.axis_index("s")
    sc_base = c * HALF

    zeros16 = jnp.zeros((L,), jnp.float32)

    @pl.loop(0, N // L)
    def _z0(k):
        hist_v[pl.ds(k * L, L)] = zeros16

    @pl.loop(0, ACC_ROWS)
    def _z1(r):
        for j in range(D // L):
            acc[r, pl.ds(j * L, L)] = zeros16

    ebase = s * EPT
    lane = lax.iota(jnp.int32, L)

    @pl.loop(0, EPT // SB)
    def _deg_chunk(cb):
        off = ebase + cb * SB
        pltpu.sync_copy(row_hbm.at[pl.ds(off, SB)], rbuf)
        pltpu.sync_copy(ev_hbm.at[pl.ds(off, SB)], ebuf)
        @pl.loop(0, SB // L)
        def _deg(k):
            sl = pl.ds(k * L, L)
            rv = rbuf[sl]
            e16 = ebuf[sl]
            for k2 in range(L):
                plsc.addupdate_scatter(hist_v, [rv], e16, mask=lane == k2)

    pltpu.sync_copy(hist_v, hist_sh.at[pl.ds(s * N, N)])
    plsc.subcore_barrier()

    @pl.loop(s, N // L, step=NS)
    def _comb(k):
        for j in range(NS):
            pltpu.sync_copy(hist_sh.at[pl.ds(j * N + k * L, L)],
                            tmp.at[pl.ds(j * L, L)])
        rs = tmp[pl.ds(0, L)]
        for j in range(1, NS):
            rs = rs + tmp[pl.ds(j * L, L)]
        pos = rs > jnp.float32(0.0)
        safe = jnp.where(pos, rs, jnp.float32(1.0))
        tmpv[pl.ds(0, L)] = jnp.where(pos, _nr_rsqrt(safe), jnp.float32(0.0))
        pltpu.sync_copy(tmpv, rs_sh.at[pl.ds(k * L, L)])
    plsc.subcore_barrier()

    pltpu.sync_copy(rs_sh, hist_v)
    dinv_v = hist_v

    def flush():
        pltpu.async_copy(feat_hbm.at[colb.at[pl.ds(0, FB)]], gbuf, sem).wait()
        @pl.loop(0, FB // L)
        def _fma(g):
            tvec = tgtb[pl.ds(g * L, L)]
            cvec = cofb[pl.ds(g * L, L)]
            for k in range(L):
                t = tvec[k]
                cs = cvec[k]
                e = g * L + k
                for j in range(D // L):
                    sl = pl.ds(j * L, L)
                    acc[t, sl] = acc[t, sl] + gbuf[e, sl] * cs

    @pl.loop(0, E // SB, init_carry=jnp.int32(0))
    def _scan_chunk(cb, cnt):
        off = cb * SB
        pltpu.sync_copy(row_hbm.at[pl.ds(off, SB)], rbuf)
        pltpu.sync_copy(col_hbm.at[pl.ds(off, SB)], cbuf)
        pltpu.sync_copy(ev_hbm.at[pl.ds(off, SB)], ebuf)

        @pl.loop(0, SB // L, init_carry=cnt)
        def _scan(k, cnt):
            sl = pl.ds(k * L, L)
            rv = rbuf[sl]
            cv = cbuf[sl]
            l = rv - sc_base
            mine = ((l >= 0) & (l < HALF)
                    & (jnp.right_shift(l, 3) % NS == s))
            tloc = jnp.left_shift(jnp.right_shift(l, 7), 3) | (l % 8)
            dr = plsc.load_gather(dinv_v, [rv])
            dc = plsc.load_gather(dinv_v, [cv])
            co = ebuf[sl] * dr * dc
            plsc.store_compressed(colb.at[pl.ds(cnt, L)], cv, mask=mine)
            plsc.store_compressed(tgtb.at[pl.ds(cnt, L)], tloc, mask=mine)
            plsc.store_compressed(cofb.at[pl.ds(cnt, L)], co, mask=mine)
            cnt = cnt + plsc.all_reduce_population_count(mine)[0]

            @pl.when(cnt >= FB)
            def _do_flush():
                flush()
                colb[pl.ds(0, L)] = colb[pl.ds(FB, L)]
                tgtb[pl.ds(0, L)] = tgtb[pl.ds(FB, L)]
                cofb[pl.ds(0, L)] = cofb[pl.ds(FB, L)]

            return jnp.where(cnt >= FB, cnt - FB, cnt)

        return _scan

    cnt = _scan_chunk
    @pl.loop(0, FB // L)
    def _pad(g):
        sl = pl.ds(g * L, L)
        idx = lax.iota(jnp.int32, L) + g * L
        keep = idx < cnt
        colb[sl] = jnp.where(keep, colb[sl], 0)
        tgtb[sl] = jnp.where(keep, tgtb[sl], 0)
        cofb[sl] = jnp.where(keep, cofb[sl], jnp.float32(0.0))
    flush()

    @pl.loop(s, NBLK8, step=NS)
    def _wb(b):
        slot = jnp.right_shift(b, 4)
        pltpu.sync_copy(acc.at[pl.ds(slot * 8, 8), :],
                        out_hbm.at[pl.ds(sc_base + b * 8, 8), :])


@jax.jit
def _gcn_sc(features, row, col, ev):
    mesh = plsc.VectorSubcoreMesh(
        core_axis_name="c", subcore_axis_name="s",
        num_cores=NC, num_subcores=NS)
    f = functools.partial(
        pl.kernel,
        out_type=jax.ShapeDtypeStruct((N, D), jnp.float32),
        mesh=mesh,
        compiler_params=pltpu.CompilerParams(needs_layout_passes=False),
        scratch_types=[
            pltpu.VMEM_SHARED((NS * N,), jnp.float32),
            pltpu.VMEM_SHARED((N,), jnp.float32),
            pltpu.VMEM((N,), jnp.float32),
            pltpu.VMEM((ACC_ROWS, D), jnp.float32),
            pltpu.VMEM((FB, D), jnp.float32),
            pltpu.VMEM((SB,), jnp.int32),
            pltpu.VMEM((SB,), jnp.int32),
            pltpu.VMEM((SB,), jnp.float32),
            pltpu.VMEM((CAP,), jnp.int32),
            pltpu.VMEM((CAP,), jnp.int32),
            pltpu.VMEM((CAP,), jnp.float32),
            pltpu.VMEM((NS * L,), jnp.float32),
            pltpu.VMEM((L,), jnp.float32),
            pltpu.SemaphoreType.DMA,
        ],
    )(_body)
    return f(features, row, col, ev)


def kernel(features, edge_index, edge_values, index):
    del index
    row = edge_index[0]
    col = edge_index[1]
    return _gcn_sc(features, row, col, edge_values)
